# baseline (speedup 1.0000x reference)
import functools

import jax
import jax.numpy as jnp
from jax import lax
from jax.experimental import pallas as pl
from jax.experimental.pallas import tpu as pltpu
from jax.experimental.pallas import tpu_sc as plsc

_INFO = plsc.get_sparse_core_info()
_NC, _NS = _INFO.num_cores, _INFO.num_subcores
_NW = _NC * _NS

_BATCH = 4
_SEQ = 4096
_D = 1024
_PR = 8
_NP = _BATCH * _SEQ * _PR
_PPW = _NP // _NW
_CP = 128
_NCHUNK = _PPW // _CP
_NBUF = 7
_LOOKAHEAD = 4


def _gather_kernel(pidx_hbm, table_hbm, out_hbm, pidx_v, bufs, gsems, wsems):
    wid = lax.axis_index("s") * _NC + lax.axis_index("c")
    base = wid * _PPW
    b = base // (_SEQ * _PR)
    off = base % (_SEQ * _PR)
    pltpu.sync_copy(pidx_hbm.at[pl.ds(base, _PPW)], pidx_v)

    def gather(c):
        return pltpu.async_copy(
            table_hbm.at[pidx_v.at[pl.ds(c * _CP, _CP)]],
            bufs[c % _NBUF], gsems[c % _NBUF])

    def write(c):
        return pltpu.async_copy(
            bufs[c % _NBUF],
            out_hbm.at[b, pl.ds(off + c * _CP, _CP)],
            wsems[c % _NBUF])

    g = [None] * _NCHUNK
    w = [None] * _NCHUNK
    for c in range(_LOOKAHEAD):
        g[c] = gather(c)
    for c in range(_NCHUNK):
        g[c].wait()
        w[c] = write(c)
        nc = c + _LOOKAHEAD
        if nc < _NCHUNK:
            if nc - _NBUF >= 0:
                w[nc - _NBUF].wait()
            g[nc] = gather(nc)
    for c in range(max(0, _NCHUNK - _NBUF), _NCHUNK):
        w[c].wait()


@jax.jit
def _run(pidx, table):
    mesh = plsc.VectorSubcoreMesh(core_axis_name="c", subcore_axis_name="s")
    k = pl.kernel(
        _gather_kernel,
        out_type=jax.ShapeDtypeStruct((_BATCH, _SEQ * _PR, 128), jnp.float32),
        mesh=mesh,
        compiler_params=pltpu.CompilerParams(use_tc_tiling_on_sc=True),
        scratch_types=[
            pltpu.VMEM((_PPW,), jnp.int32),
            [pltpu.VMEM((_CP, 128), jnp.float32) for _ in range(_NBUF)],
            [pltpu.SemaphoreType.DMA for _ in range(_NBUF)],
            [pltpu.SemaphoreType.DMA for _ in range(_NBUF)],
        ],
    )
    return k(pidx, table)


def kernel(accumulated_times, time_encoding):
    pidx = (accumulated_times.reshape(-1)[:, None] * _PR
            + jnp.arange(_PR, dtype=jnp.int32)).reshape(-1)
    table = time_encoding.reshape(time_encoding.shape[1] * _PR, 128)
    out = _run(pidx, table)
    return out.reshape(_BATCH, _SEQ, _D)


# speedup vs baseline: 1.9413x; 1.9413x over previous
import functools

import jax
import jax.numpy as jnp
from jax import lax
from jax.experimental import pallas as pl
from jax.experimental.pallas import tpu as pltpu
from jax.experimental.pallas import tpu_sc as plsc

_INFO = plsc.get_sparse_core_info()
_NC, _NS = _INFO.num_cores, _INFO.num_subcores
_NW = _NC * _NS

_BATCH = 4
_SEQ = 4096
_D = 1024
_BPW = _BATCH * _SEQ // _NW
_WPB = _SEQ // _BPW
_C = 16
_NCHUNK = _BPW // _C
_NBUF = 7
_LOOKAHEAD = 5


def _gather_kernel(idx_hbm, table_hbm, out_hbm, idx_v, bufs, gsems, wsems):
    wid = lax.axis_index("s") * _NC + lax.axis_index("c")
    b = wid // _WPB
    off = (wid % _WPB) * _BPW
    pltpu.sync_copy(idx_hbm.at[b, pl.ds(off, _BPW)], idx_v)
    table2d = table_hbm.at[0]

    def gather(c):
        return pltpu.async_copy(
            table2d.at[idx_v.at[pl.ds(c * _C, _C)]],
            bufs[c % _NBUF], gsems[c % _NBUF])

    def write(c):
        return pltpu.async_copy(
            bufs[c % _NBUF], out_hbm.at[b, pl.ds(off + c * _C, _C)],
            wsems[c % _NBUF])

    g = [None] * _NCHUNK
    w = [None] * _NCHUNK
    for c in range(_LOOKAHEAD):
        g[c] = gather(c)
    for c in range(_NCHUNK):
        g[c].wait()
        w[c] = write(c)
        nc = c + _LOOKAHEAD
        if nc < _NCHUNK:
            if nc - _NBUF >= 0:
                w[nc - _NBUF].wait()
            g[nc] = gather(nc)
    for c in range(max(0, _NCHUNK - _NBUF), _NCHUNK):
        w[c].wait()


@jax.jit
def _run(idx, table):
    mesh = plsc.VectorSubcoreMesh(core_axis_name="c", subcore_axis_name="s")
    k = pl.kernel(
        _gather_kernel,
        out_type=jax.ShapeDtypeStruct((_BATCH, _SEQ, _D), jnp.float32),
        mesh=mesh,
        compiler_params=pltpu.CompilerParams(use_tc_tiling_on_sc=True),
        scratch_types=[
            pltpu.VMEM((_BPW,), jnp.int32),
            [pltpu.VMEM((_C, _D), jnp.float32) for _ in range(_NBUF)],
            [pltpu.SemaphoreType.DMA for _ in range(_NBUF)],
            [pltpu.SemaphoreType.DMA for _ in range(_NBUF)],
        ],
    )
    return k(idx, table)


def kernel(accumulated_times, time_encoding):
    return _run(accumulated_times, time_encoding)
